# Initial kernel scaffold; baseline (speedup 1.0000x reference)
import jax
import jax.numpy as jnp
from jax.experimental import pallas as pl


def kernel(attn_raw):
    raise NotImplementedError("write your pallas kernel here")


# speedup vs baseline: 4.0110x; 4.0110x over previous
import functools

import jax
import jax.numpy as jnp
from jax import lax
from jax.experimental import pallas as pl
from jax.experimental.pallas import tpu as pltpu
from jax.experimental.pallas import tpu_sc as plsc

_EPS = 1e-7
_ROWS = 128
_COLS = 32768
_LANES = 16
_NBLK = _COLS // _LANES
_NWORKERS = 32
_ROWS_PER_W = _ROWS // _NWORKERS
_NEG_INF = float("-inf")


def _sortd(v):
    k, _ = plsc.sort_key_val(v, v, descending=True)
    return k


def _tophalf(a, b):
    return _sortd(jnp.maximum(a, lax.rev(b, (0,))))


def _bothhalves(a, b):
    br = lax.rev(b, (0,))
    return _sortd(jnp.maximum(a, br)), _sortd(jnp.minimum(a, br))


_GATHER_DN = lax.GatherDimensionNumbers(
    offset_dims=(), collapsed_slice_dims=(0,), start_index_map=(0,))


def _bcast_last(v):
    idx = jnp.full((_LANES, 1), _LANES - 1, jnp.int32)
    return lax.gather(v, idx, _GATHER_DN, (1,),
                      mode=lax.GatherScatterMode.PROMISE_IN_BOUNDS)


@functools.partial(
    pl.kernel,
    out_type=jax.ShapeDtypeStruct((_ROWS, _LANES), jnp.float32),
    mesh=plsc.VectorSubcoreMesh(core_axis_name="c", subcore_axis_name="s"),
    compiler_params=pltpu.CompilerParams(needs_layout_passes=False),
    scratch_types=[
        pltpu.VMEM((_COLS,), jnp.float32),
        pltpu.VMEM((_LANES,), jnp.float32),
    ],
)
def _sc_topk(x_hbm, t_hbm, row_buf, tvec_buf):
    wid = lax.axis_index("s") * 2 + lax.axis_index("c")

    def do_row(r, carry):
        row = wid * _ROWS_PER_W + r
        pltpu.sync_copy(x_hbm.at[row], row_buf)

        def pass_a(i, ms):
            m0, m1 = ms
            x0 = row_buf[pl.ds(i * 32, _LANES)]
            x1 = row_buf[pl.ds(i * 32 + _LANES, _LANES)]
            return jnp.maximum(m0, x0), jnp.maximum(m1, x1)

        ninf = jnp.full((_LANES,), _NEG_INF, jnp.float32)
        m0, m1 = lax.fori_loop(0, _COLS // 32, pass_a, (ninf, ninf))
        g = _bcast_last(_sortd(jnp.minimum(m0, m1)))

        def pass_b(i, st):
            s0, s1, thr = st
            x = row_buf[pl.ds(i * _LANES, _LANES)]
            hit = jnp.any(x >= thr)

            def merge(op):
                s0, s1, _ = op
                xs = _sortd(x)
                h = _tophalf(xs, s1)
                ns0, ns1 = _bothhalves(s0, h)
                return ns0, ns1, jnp.maximum(g, _bcast_last(ns1))

            return lax.cond(hit, merge, lambda op: op, (s0, s1, thr))

        s0, s1, thr = lax.fori_loop(0, _NBLK, pass_b, (ninf, ninf, g))
        tvec_buf[...] = _bcast_last(s1)
        pltpu.sync_copy(tvec_buf, t_hbm.at[row])
        return carry

    lax.fori_loop(0, _ROWS_PER_W, do_row, 0)


_TC_BLOCK = 8


def _tc_norm_body(x_ref, t_ref, o_ref):
    x = x_ref[...]
    t = t_ref[...][:, :1]
    w = jnp.maximum(x - (t + _EPS), 0.0)
    s = jnp.sum(w, axis=1, keepdims=True) + _EPS
    o_ref[...] = w / s


def kernel(attn_raw):
    t = _sc_topk(attn_raw)
    out = pl.pallas_call(
        _tc_norm_body,
        grid=(_ROWS // _TC_BLOCK,),
        in_specs=[
            pl.BlockSpec((_TC_BLOCK, _COLS), lambda i: (i, 0)),
            pl.BlockSpec((_TC_BLOCK, _LANES), lambda i: (i, 0)),
        ],
        out_specs=pl.BlockSpec((_TC_BLOCK, _COLS), lambda i: (i, 0)),
        out_shape=jax.ShapeDtypeStruct((_ROWS, _COLS), jnp.float32),
    )(attn_raw, t)
    return out


# speedup vs baseline: 9.1652x; 2.2850x over previous
import functools

import jax
import jax.numpy as jnp
from jax import lax
from jax.experimental import pallas as pl
from jax.experimental.pallas import tpu as pltpu
from jax.experimental.pallas import tpu_sc as plsc

_EPS = 1e-7
_ROWS = 128
_COLS = 32768
_LANES = 16
_NBLK = _COLS // _LANES
_NWORKERS = 32
_ROWS_PER_W = _ROWS // _NWORKERS
_NEG_INF = float("-inf")


def _sortd(v):
    k, _ = plsc.sort_key_val(v, v, descending=True)
    return k


def _tophalf(a, b):
    return _sortd(jnp.maximum(a, lax.rev(b, (0,))))


def _bothhalves(a, b):
    br = lax.rev(b, (0,))
    return _sortd(jnp.maximum(a, br)), _sortd(jnp.minimum(a, br))


_GATHER_DN = lax.GatherDimensionNumbers(
    offset_dims=(), collapsed_slice_dims=(0,), start_index_map=(0,))


def _bcast_last(v):
    idx = jnp.full((_LANES, 1), _LANES - 1, jnp.int32)
    return lax.gather(v, idx, _GATHER_DN, (1,),
                      mode=lax.GatherScatterMode.PROMISE_IN_BOUNDS)


_SG = 8
_NSG = _NBLK // _SG


def _merge_block(x, st, g):
    s0, s1, _ = st
    xs = _sortd(x)
    h = _tophalf(xs, s1)
    ns0, ns1 = _bothhalves(s0, h)
    return ns0, ns1, jnp.maximum(g, _bcast_last(ns1))


@functools.partial(
    pl.kernel,
    out_type=jax.ShapeDtypeStruct((_ROWS, _LANES), jnp.float32),
    mesh=plsc.VectorSubcoreMesh(core_axis_name="c", subcore_axis_name="s"),
    compiler_params=pltpu.CompilerParams(needs_layout_passes=False),
    scratch_types=[
        pltpu.VMEM((_COLS,), jnp.float32),
        pltpu.VMEM((_NSG * _LANES,), jnp.float32),
        pltpu.VMEM((_LANES,), jnp.float32),
    ],
)
def _sc_topk(x_hbm, t_hbm, row_buf, sgmax_buf, tvec_buf):
    wid = lax.axis_index("s") * 2 + lax.axis_index("c")

    def do_row(r, carry):
        row = wid * _ROWS_PER_W + r
        pltpu.sync_copy(x_hbm.at[row], row_buf)

        def pass_a(i, ms):
            m0, m1 = ms
            base = i * (_SG * _LANES)
            v = [row_buf[pl.ds(base + j * _LANES, _LANES)]
                 for j in range(_SG)]
            e = jnp.maximum(jnp.maximum(v[0], v[2]), jnp.maximum(v[4], v[6]))
            o = jnp.maximum(jnp.maximum(v[1], v[3]), jnp.maximum(v[5], v[7]))
            sgmax_buf[pl.ds(i * _LANES, _LANES)] = jnp.maximum(e, o)
            return jnp.maximum(m0, e), jnp.maximum(m1, o)

        ninf = jnp.full((_LANES,), _NEG_INF, jnp.float32)
        m0, m1 = lax.fori_loop(0, _NSG, pass_a, (ninf, ninf))
        g = _bcast_last(_sortd(jnp.minimum(m0, m1)))

        def pass_b(i, st):
            sgm = sgmax_buf[pl.ds(i * _LANES, _LANES)]
            hit = jnp.any(sgm >= st[2])

            def descend(op):
                base = i * (_SG * _LANES)
                for j in range(_SG):
                    x = row_buf[pl.ds(base + j * _LANES, _LANES)]
                    hit_j = jnp.any(x >= op[2])
                    op = lax.cond(
                        hit_j, lambda o, xx=x: _merge_block(xx, o, g),
                        lambda o: o, op)
                return op

            return lax.cond(hit, descend, lambda op: op, st)

        s0, s1, thr = lax.fori_loop(0, _NSG, pass_b, (ninf, ninf, g))
        tvec_buf[...] = _bcast_last(s1)
        pltpu.sync_copy(tvec_buf, t_hbm.at[row])
        return carry

    lax.fori_loop(0, _ROWS_PER_W, do_row, 0)


_TC_BLOCK = 8


def _tc_norm_body(x_ref, t_ref, o_ref):
    x = x_ref[...]
    t = t_ref[...][:, :1]
    w = jnp.maximum(x - (t + _EPS), 0.0)
    s = jnp.sum(w, axis=1, keepdims=True) + _EPS
    o_ref[...] = w / s


def kernel(attn_raw):
    t = _sc_topk(attn_raw)
    out = pl.pallas_call(
        _tc_norm_body,
        grid=(_ROWS // _TC_BLOCK,),
        in_specs=[
            pl.BlockSpec((_TC_BLOCK, _COLS), lambda i: (i, 0)),
            pl.BlockSpec((_TC_BLOCK, _LANES), lambda i: (i, 0)),
        ],
        out_specs=pl.BlockSpec((_TC_BLOCK, _COLS), lambda i: (i, 0)),
        out_shape=jax.ShapeDtypeStruct((_ROWS, _COLS), jnp.float32),
    )(attn_raw, t)
    return out


# speedup vs baseline: 11.3308x; 1.2363x over previous
import functools

import jax
import jax.numpy as jnp
from jax import lax
from jax.experimental import pallas as pl
from jax.experimental.pallas import tpu as pltpu
from jax.experimental.pallas import tpu_sc as plsc

_EPS = 1e-7
_ROWS = 128
_COLS = 32768
_LANES = 16
_NBLK = _COLS // _LANES
_NWORKERS = 32
_ROWS_PER_W = _ROWS // _NWORKERS
_NEG_INF = float("-inf")


def _sortd(v):
    k, _ = plsc.sort_key_val(v, v, descending=True)
    return k


def _tophalf(a, b):
    return _sortd(jnp.maximum(a, lax.rev(b, (0,))))


def _bothhalves(a, b):
    br = lax.rev(b, (0,))
    return _sortd(jnp.maximum(a, br)), _sortd(jnp.minimum(a, br))


_GATHER_DN = lax.GatherDimensionNumbers(
    offset_dims=(), collapsed_slice_dims=(0,), start_index_map=(0,))


def _bcast_last(v):
    idx = jnp.full((_LANES, 1), _LANES - 1, jnp.int32)
    return lax.gather(v, idx, _GATHER_DN, (1,),
                      mode=lax.GatherScatterMode.PROMISE_IN_BOUNDS)


_SG = 8
_NSG = _NBLK // _SG


def _merge_block(x, s0, s1):
    h = _tophalf(x, s1)
    return _bothhalves(s0, h)


@functools.partial(
    pl.kernel,
    out_type=jax.ShapeDtypeStruct((_ROWS, _LANES), jnp.float32),
    mesh=plsc.VectorSubcoreMesh(core_axis_name="c", subcore_axis_name="s"),
    compiler_params=pltpu.CompilerParams(needs_layout_passes=False),
    scratch_types=[
        pltpu.VMEM((_COLS,), jnp.float32),
        pltpu.VMEM((_COLS + _LANES,), jnp.float32),
        pltpu.VMEM((_LANES,), jnp.float32),
    ],
)
def _sc_topk(x_hbm, t_hbm, row_buf, cand_buf, tvec_buf):
    wid = lax.axis_index("s") * 2 + lax.axis_index("c")
    lane_iota = lax.iota(jnp.int32, _LANES)

    def do_row(r, carry):
        row = wid * _ROWS_PER_W + r
        pltpu.sync_copy(x_hbm.at[row], row_buf)

        def pass_a(i, ms):
            m0, m1 = ms
            base = i * (_SG * _LANES)
            v = [row_buf[pl.ds(base + j * _LANES, _LANES)]
                 for j in range(_SG)]
            e = jnp.maximum(jnp.maximum(v[0], v[2]), jnp.maximum(v[4], v[6]))
            o = jnp.maximum(jnp.maximum(v[1], v[3]), jnp.maximum(v[5], v[7]))
            return jnp.maximum(m0, e), jnp.maximum(m1, o)

        ninf = jnp.full((_LANES,), _NEG_INF, jnp.float32)
        m0, m1 = lax.fori_loop(0, _NSG, pass_a, (ninf, ninf))
        g = _bcast_last(_sortd(jnp.minimum(m0, m1)))

        def pass_b(i, coff):
            base = i * (_SG * _LANES)
            for j in range(_SG):
                x = row_buf[pl.ds(base + j * _LANES, _LANES)]
                m = x >= g
                pos = plsc.cumsum(jnp.where(m, 1, 0).astype(jnp.int32))
                plsc.store_scatter(cand_buf, [coff + pos - 1], x, mask=m)
                coff = coff + plsc.all_reduce_population_count(m)
            return coff

        zero = jnp.zeros((_LANES,), jnp.int32)
        coff = lax.fori_loop(0, _NSG, pass_b, zero)
        ncand = jnp.max(coff)
        nfull = ncand // _LANES

        def pass_c(k, st):
            s0, s1 = st
            x = _sortd(cand_buf[pl.ds(k * _LANES, _LANES)])
            return _merge_block(x, s0, s1)

        s0, s1 = lax.fori_loop(0, nfull, pass_c, (ninf, ninf))
        xt = cand_buf[pl.ds(nfull * _LANES, _LANES)]
        xt = jnp.where(lane_iota < (ncand - nfull * _LANES), xt, ninf)
        s0, s1 = _merge_block(_sortd(xt), s0, s1)

        tvec_buf[...] = _bcast_last(s1)
        pltpu.sync_copy(tvec_buf, t_hbm.at[row])
        return carry

    lax.fori_loop(0, _ROWS_PER_W, do_row, 0)


_TC_BLOCK = 8


def _tc_norm_body(x_ref, t_ref, o_ref):
    x = x_ref[...]
    t = t_ref[...][:, :1]
    w = jnp.maximum(x - (t + _EPS), 0.0)
    s = jnp.sum(w, axis=1, keepdims=True) + _EPS
    o_ref[...] = w / s


def kernel(attn_raw):
    t = _sc_topk(attn_raw)
    out = pl.pallas_call(
        _tc_norm_body,
        grid=(_ROWS // _TC_BLOCK,),
        in_specs=[
            pl.BlockSpec((_TC_BLOCK, _COLS), lambda i: (i, 0)),
            pl.BlockSpec((_TC_BLOCK, _LANES), lambda i: (i, 0)),
        ],
        out_specs=pl.BlockSpec((_TC_BLOCK, _COLS), lambda i: (i, 0)),
        out_shape=jax.ShapeDtypeStruct((_ROWS, _COLS), jnp.float32),
    )(attn_raw, t)
    return out


# speedup vs baseline: 28.2363x; 2.4920x over previous
import functools

import jax
import jax.numpy as jnp
from jax import lax
from jax.experimental import pallas as pl
from jax.experimental.pallas import tpu as pltpu
from jax.experimental.pallas import tpu_sc as plsc

_EPS = 1e-7
_ROWS = 128
_COLS = 32768
_LANES = 16
_NBLK = _COLS // _LANES
_NWORKERS = 32
_ROWS_PER_W = _ROWS // _NWORKERS
_NEG_INF = float("-inf")


def _sortd(v):
    k, _ = plsc.sort_key_val(v, v, descending=True)
    return k


def _tophalf(a, b):
    return _sortd(jnp.maximum(a, lax.rev(b, (0,))))


def _bothhalves(a, b):
    br = lax.rev(b, (0,))
    return _sortd(jnp.maximum(a, br)), _sortd(jnp.minimum(a, br))


_GATHER_DN = lax.GatherDimensionNumbers(
    offset_dims=(), collapsed_slice_dims=(0,), start_index_map=(0,))


def _bcast_last(v):
    idx = jnp.full((_LANES, 1), _LANES - 1, jnp.int32)
    return lax.gather(v, idx, _GATHER_DN, (1,),
                      mode=lax.GatherScatterMode.PROMISE_IN_BOUNDS)


_SG = 8
_NSG = _NBLK // _SG


def _merge_block(x, s0, s1):
    h = _tophalf(x, s1)
    return _bothhalves(s0, h)


@functools.partial(
    pl.kernel,
    out_type=jax.ShapeDtypeStruct((_ROWS, _LANES), jnp.float32),
    mesh=plsc.VectorSubcoreMesh(core_axis_name="c", subcore_axis_name="s"),
    compiler_params=pltpu.CompilerParams(needs_layout_passes=False),
    scratch_types=[
        pltpu.VMEM((_COLS,), jnp.float32),
        pltpu.VMEM((_COLS,), jnp.float32),
        pltpu.VMEM((_COLS + _LANES,), jnp.float32),
        pltpu.VMEM((_NSG * _LANES,), jnp.float32),
        pltpu.VMEM((_NSG + _LANES,), jnp.int32),
        pltpu.VMEM((_LANES,), jnp.float32),
        pltpu.SemaphoreType.DMA,
        pltpu.SemaphoreType.DMA,
    ],
)
def _sc_topk(x_hbm, t_hbm, row_a, row_b, cand_buf, sgmax_buf, wl_buf,
             tvec_buf, sem_a, sem_b):
    wid = lax.axis_index("s") * 2 + lax.axis_index("c")
    lane_iota = lax.iota(jnp.int32, _LANES)
    row0 = wid * _ROWS_PER_W
    bufs = [row_a, row_b]
    sems = [sem_a, sem_b]
    ninf = jnp.full((_LANES,), _NEG_INF, jnp.float32)
    zero = jnp.zeros((_LANES,), jnp.int32)

    def process(row, row_buf):
        def pass_a(i, ms):
            m0, m1 = ms
            base = i * (_SG * _LANES)
            v = [row_buf[pl.ds(base + j * _LANES, _LANES)]
                 for j in range(_SG)]
            e = jnp.maximum(jnp.maximum(v[0], v[2]), jnp.maximum(v[4], v[6]))
            o = jnp.maximum(jnp.maximum(v[1], v[3]), jnp.maximum(v[5], v[7]))
            sgmax_buf[pl.ds(i * _LANES, _LANES)] = jnp.maximum(e, o)
            return jnp.maximum(m0, e), jnp.maximum(m1, o)

        m0, m1 = lax.fori_loop(0, _NSG, pass_a, (ninf, ninf))
        g = _bcast_last(_sortd(jnp.minimum(m0, m1)))

        woff = zero
        for v in range(_NSG // _LANES):
            sgm = sgmax_buf[pl.ds(v * _LANES, _LANES)]
            m = sgm >= g
            pos = plsc.cumsum(jnp.where(m, 1, 0).astype(jnp.int32))
            plsc.store_scatter(wl_buf, [woff + pos - 1],
                               lane_iota + v * _LANES, mask=m)
            woff = woff + plsc.all_reduce_population_count(m)
        n_sg = jnp.max(woff)

        def pass_b(i, coff):
            wlv = wl_buf[pl.ds(i, _LANES)]
            base = wlv[0] * (_SG * _LANES)
            for j in range(_SG):
                x = row_buf[pl.ds(base + j * _LANES, _LANES)]
                m = x >= g
                pos = plsc.cumsum(jnp.where(m, 1, 0).astype(jnp.int32))
                plsc.store_scatter(cand_buf, [coff + pos - 1], x, mask=m)
                coff = coff + plsc.all_reduce_population_count(m)
            return coff

        coff = lax.fori_loop(0, n_sg, pass_b, zero)
        ncand = jnp.max(coff)
        nfull = ncand // _LANES

        def pass_c(k, st):
            s0, s1 = st
            x = _sortd(cand_buf[pl.ds(k * _LANES, _LANES)])
            return _merge_block(x, s0, s1)

        s0, s1 = lax.fori_loop(0, nfull, pass_c, (ninf, ninf))
        xt = cand_buf[pl.ds(nfull * _LANES, _LANES)]
        xt = jnp.where(lane_iota < (ncand - nfull * _LANES), xt, ninf)
        s0, s1 = _merge_block(_sortd(xt), s0, s1)

        tvec_buf[...] = _bcast_last(s1)
        pltpu.sync_copy(tvec_buf, t_hbm.at[row])

    cp = pltpu.async_copy(x_hbm.at[row0], bufs[0], sems[0])
    for r in range(_ROWS_PER_W):
        nxt = None
        if r + 1 < _ROWS_PER_W:
            nxt = pltpu.async_copy(
                x_hbm.at[row0 + r + 1], bufs[(r + 1) % 2], sems[(r + 1) % 2])
        cp.wait()
        process(row0 + r, bufs[r % 2])
        cp = nxt


_TC_BLOCK = 8


def _tc_norm_body(x_ref, t_ref, o_ref):
    x = x_ref[...]
    t = t_ref[...][:, :1]
    w = jnp.maximum(x - (t + _EPS), 0.0)
    s = jnp.sum(w, axis=1, keepdims=True) + _EPS
    o_ref[...] = w / s


def kernel(attn_raw):
    t = _sc_topk(attn_raw)
    out = pl.pallas_call(
        _tc_norm_body,
        grid=(_ROWS // _TC_BLOCK,),
        in_specs=[
            pl.BlockSpec((_TC_BLOCK, _COLS), lambda i: (i, 0)),
            pl.BlockSpec((_TC_BLOCK, _LANES), lambda i: (i, 0)),
        ],
        out_specs=pl.BlockSpec((_TC_BLOCK, _COLS), lambda i: (i, 0)),
        out_shape=jax.ShapeDtypeStruct((_ROWS, _COLS), jnp.float32),
    )(attn_raw, t)
    return out


# speedup vs baseline: 31.1963x; 1.1048x over previous
import functools

import jax
import jax.numpy as jnp
from jax import lax
from jax.experimental import pallas as pl
from jax.experimental.pallas import tpu as pltpu
from jax.experimental.pallas import tpu_sc as plsc

_EPS = 1e-7
_ROWS = 128
_COLS = 32768
_LANES = 16
_NBLK = _COLS // _LANES
_NWORKERS = 32
_ROWS_PER_W = _ROWS // _NWORKERS
_NEG_INF = float("-inf")


def _sortd(v):
    k, _ = plsc.sort_key_val(v, v, descending=True)
    return k


def _tophalf(a, b):
    return _sortd(jnp.maximum(a, lax.rev(b, (0,))))


def _bothhalves(a, b):
    br = lax.rev(b, (0,))
    return _sortd(jnp.maximum(a, br)), _sortd(jnp.minimum(a, br))


_GATHER_DN = lax.GatherDimensionNumbers(
    offset_dims=(), collapsed_slice_dims=(0,), start_index_map=(0,))


def _bcast_last(v):
    idx = jnp.full((_LANES, 1), _LANES - 1, jnp.int32)
    return lax.gather(v, idx, _GATHER_DN, (1,),
                      mode=lax.GatherScatterMode.PROMISE_IN_BOUNDS)


_SG = 8
_NSG = _NBLK // _SG


def _merge_block(x, s0, s1):
    h = _tophalf(x, s1)
    return _bothhalves(s0, h)


@functools.partial(
    pl.kernel,
    out_type=jax.ShapeDtypeStruct((_ROWS, _LANES), jnp.float32),
    mesh=plsc.VectorSubcoreMesh(core_axis_name="c", subcore_axis_name="s"),
    compiler_params=pltpu.CompilerParams(needs_layout_passes=False),
    scratch_types=[
        pltpu.VMEM((_COLS,), jnp.float32),
        pltpu.VMEM((_COLS,), jnp.float32),
        pltpu.VMEM((_COLS + _LANES,), jnp.float32),
        pltpu.VMEM((_NSG * _LANES,), jnp.float32),
        pltpu.VMEM((_NSG + _LANES,), jnp.int32),
        pltpu.VMEM((_LANES,), jnp.float32),
        pltpu.SemaphoreType.DMA,
        pltpu.SemaphoreType.DMA,
    ],
)
def _sc_topk(x_hbm, t_hbm, row_a, row_b, cand_buf, sgmax_buf, wl_buf,
             tvec_buf, sem_a, sem_b):
    wid = lax.axis_index("s") * 2 + lax.axis_index("c")
    lane_iota = lax.iota(jnp.int32, _LANES)
    row0 = wid * _ROWS_PER_W
    bufs = [row_a, row_b]
    sems = [sem_a, sem_b]
    ninf = jnp.full((_LANES,), _NEG_INF, jnp.float32)
    zero = jnp.zeros((_LANES,), jnp.int32)

    def process(row, row_buf):
        def pass_a(i, ms):
            m0, m1 = ms
            base = i * (_SG * _LANES)
            v = [row_buf[pl.ds(base + j * _LANES, _LANES)]
                 for j in range(_SG)]
            e = jnp.maximum(jnp.maximum(v[0], v[2]), jnp.maximum(v[4], v[6]))
            o = jnp.maximum(jnp.maximum(v[1], v[3]), jnp.maximum(v[5], v[7]))
            sgmax_buf[pl.ds(i * _LANES, _LANES)] = jnp.maximum(e, o)
            return jnp.maximum(m0, e), jnp.maximum(m1, o)

        m0, m1 = lax.fori_loop(0, _NSG, pass_a, (ninf, ninf))
        g = _bcast_last(_sortd(jnp.minimum(m0, m1)))

        woff = zero
        for v in range(_NSG // _LANES):
            sgm = sgmax_buf[pl.ds(v * _LANES, _LANES)]
            m = sgm >= g
            pos = plsc.cumsum(jnp.where(m, 1, 0).astype(jnp.int32))
            plsc.store_scatter(wl_buf, [woff + pos - 1],
                               lane_iota + v * _LANES, mask=m)
            woff = woff + plsc.all_reduce_population_count(m)
        n_sg = jnp.max(woff)

        def pass_b(i, coff):
            wlv = wl_buf[pl.ds(i, _LANES)]
            base = wlv[0] * (_SG * _LANES)
            for j in range(_SG):
                x = row_buf[pl.ds(base + j * _LANES, _LANES)]
                m = x >= g
                pos = plsc.cumsum(jnp.where(m, 1, 0).astype(jnp.int32))
                plsc.store_scatter(cand_buf, [coff + pos - 1], x, mask=m)
                coff = coff + plsc.all_reduce_population_count(m)
            return coff

        coff = lax.fori_loop(0, n_sg, pass_b, zero)
        ncand = jnp.max(coff)
        nfull = ncand // _LANES

        def pass_c(k, st):
            s0, s1 = st
            x = _sortd(cand_buf[pl.ds(k * _LANES, _LANES)])
            return _merge_block(x, s0, s1)

        s0, s1 = lax.fori_loop(0, nfull, pass_c, (ninf, ninf))
        xt = cand_buf[pl.ds(nfull * _LANES, _LANES)]
        xt = jnp.where(lane_iota < (ncand - nfull * _LANES), xt, ninf)
        s0, s1 = _merge_block(_sortd(xt), s0, s1)

        dv = _bcast_last(s1) + jnp.full((_LANES,), _EPS, jnp.float32)

        def sum_c(k, acc):
            c = cand_buf[pl.ds(k * _LANES, _LANES)]
            return acc + jnp.maximum(c - dv, 0.0)

        acc = lax.fori_loop(0, nfull, sum_c,
                            jnp.zeros((_LANES,), jnp.float32))
        acc = acc + jnp.maximum(xt - dv, 0.0)
        tot = _bcast_last(plsc.cumsum(acc))
        inv = jnp.full((_LANES,), 1.0, jnp.float32) / (
            tot + jnp.full((_LANES,), _EPS, jnp.float32))

        tvec_buf[...] = jnp.where(lane_iota == 0, dv, inv)
        pltpu.sync_copy(tvec_buf, t_hbm.at[row])

    cp = pltpu.async_copy(x_hbm.at[row0], bufs[0], sems[0])
    for r in range(_ROWS_PER_W):
        nxt = None
        if r + 1 < _ROWS_PER_W:
            nxt = pltpu.async_copy(
                x_hbm.at[row0 + r + 1], bufs[(r + 1) % 2], sems[(r + 1) % 2])
        cp.wait()
        process(row0 + r, bufs[r % 2])
        cp = nxt


_TC_BLOCK = 16


def _tc_norm_body(x_ref, t_ref, o_ref):
    x = x_ref[...]
    t = t_ref[...]
    d = t[:, :1]
    inv = t[:, 1:2]
    o_ref[...] = jnp.maximum(x - d, 0.0) * inv


def kernel(attn_raw):
    t = _sc_topk(attn_raw)
    out = pl.pallas_call(
        _tc_norm_body,
        grid=(_ROWS // _TC_BLOCK,),
        in_specs=[
            pl.BlockSpec((_TC_BLOCK, _COLS), lambda i: (i, 0)),
            pl.BlockSpec((_TC_BLOCK, _LANES), lambda i: (i, 0)),
        ],
        out_specs=pl.BlockSpec((_TC_BLOCK, _COLS), lambda i: (i, 0)),
        out_shape=jax.ShapeDtypeStruct((_ROWS, _COLS), jnp.float32),
    )(attn_raw, t)
    return out


# speedup vs baseline: 31.6623x; 1.0149x over previous
import functools

import jax
import jax.numpy as jnp
from jax import lax
from jax.experimental import pallas as pl
from jax.experimental.pallas import tpu as pltpu
from jax.experimental.pallas import tpu_sc as plsc

_EPS = 1e-7
_ROWS = 128
_COLS = 32768
_LANES = 16
_NBLK = _COLS // _LANES
_NWORKERS = 32
_ROWS_PER_W = _ROWS // _NWORKERS
_NEG_INF = float("-inf")


def _sortd(v):
    k, _ = plsc.sort_key_val(v, v, descending=True)
    return k


def _tophalf(a, b):
    return _sortd(jnp.maximum(a, lax.rev(b, (0,))))


def _bothhalves(a, b):
    br = lax.rev(b, (0,))
    return _sortd(jnp.maximum(a, br)), _sortd(jnp.minimum(a, br))


_GATHER_DN = lax.GatherDimensionNumbers(
    offset_dims=(), collapsed_slice_dims=(0,), start_index_map=(0,))


def _bcast_last(v):
    idx = jnp.full((_LANES, 1), _LANES - 1, jnp.int32)
    return lax.gather(v, idx, _GATHER_DN, (1,),
                      mode=lax.GatherScatterMode.PROMISE_IN_BOUNDS)


_SG = 8
_NSG = _NBLK // _SG


def _merge_block(x, s0, s1):
    h = _tophalf(x, s1)
    return _bothhalves(s0, h)


@functools.partial(
    pl.kernel,
    out_type=jax.ShapeDtypeStruct((_ROWS, _LANES), jnp.float32),
    mesh=plsc.VectorSubcoreMesh(core_axis_name="c", subcore_axis_name="s"),
    compiler_params=pltpu.CompilerParams(needs_layout_passes=False),
    scratch_types=[
        pltpu.VMEM((_COLS,), jnp.float32),
        pltpu.VMEM((_COLS,), jnp.float32),
        pltpu.VMEM((_COLS + _LANES,), jnp.float32),
        pltpu.VMEM((_NSG * _LANES,), jnp.float32),
        pltpu.VMEM((_NSG + _LANES,), jnp.int32),
        pltpu.VMEM((_LANES,), jnp.float32),
        pltpu.SemaphoreType.DMA,
        pltpu.SemaphoreType.DMA,
    ],
)
def _sc_topk(x_hbm, t_hbm, row_a, row_b, cand_buf, sgmax_buf, wl_buf,
             tvec_buf, sem_a, sem_b):
    wid = lax.axis_index("s") * 2 + lax.axis_index("c")
    lane_iota = lax.iota(jnp.int32, _LANES)
    row0 = wid * _ROWS_PER_W
    bufs = [row_a, row_b]
    sems = [sem_a, sem_b]
    ninf = jnp.full((_LANES,), _NEG_INF, jnp.float32)
    zero = jnp.zeros((_LANES,), jnp.int32)

    def process(row, row_buf):
        def pass_a(i, ms):
            m0, m1 = ms
            base = i * (_SG * _LANES)
            v = [row_buf[pl.ds(base + j * _LANES, _LANES)]
                 for j in range(_SG)]
            e = jnp.maximum(jnp.maximum(v[0], v[2]), jnp.maximum(v[4], v[6]))
            o = jnp.maximum(jnp.maximum(v[1], v[3]), jnp.maximum(v[5], v[7]))
            sgmax_buf[pl.ds(i * _LANES, _LANES)] = jnp.maximum(e, o)
            return jnp.maximum(m0, e), jnp.maximum(m1, o)

        m0, m1 = lax.fori_loop(0, _NSG, pass_a, (ninf, ninf))
        g = _bcast_last(_sortd(jnp.minimum(m0, m1)))

        woff = zero
        for v in range(_NSG // _LANES):
            sgm = sgmax_buf[pl.ds(v * _LANES, _LANES)]
            m = sgm >= g
            pos = plsc.cumsum(jnp.where(m, 1, 0).astype(jnp.int32))
            plsc.store_scatter(wl_buf, [woff + pos - 1],
                               lane_iota + v * _LANES, mask=m)
            woff = woff + plsc.all_reduce_population_count(m)
        n_sg = jnp.max(woff)

        def pass_b(i, coff):
            wlv = wl_buf[pl.ds(i, _LANES)]
            base = wlv[0] * (_SG * _LANES)
            for j in range(_SG):
                x = row_buf[pl.ds(base + j * _LANES, _LANES)]
                m = x >= g
                pos = plsc.cumsum(jnp.where(m, 1, 0).astype(jnp.int32))
                plsc.store_scatter(cand_buf, [coff + pos - 1], x, mask=m)
                coff = coff + plsc.all_reduce_population_count(m)
            return coff

        coff = lax.fori_loop(0, n_sg, pass_b, zero)
        ncand = jnp.max(coff)
        nfull = ncand // _LANES

        def pass_c(k, st):
            s0, s1 = st
            x = _sortd(cand_buf[pl.ds(k * _LANES, _LANES)])
            return _merge_block(x, s0, s1)

        s0, s1 = lax.fori_loop(0, nfull, pass_c, (ninf, ninf))
        xt = cand_buf[pl.ds(nfull * _LANES, _LANES)]
        xt = jnp.where(lane_iota < (ncand - nfull * _LANES), xt, ninf)
        s0, s1 = _merge_block(_sortd(xt), s0, s1)

        dv = _bcast_last(s1) + jnp.full((_LANES,), _EPS, jnp.float32)

        def sum_c(k, acc):
            c = cand_buf[pl.ds(k * _LANES, _LANES)]
            return acc + jnp.maximum(c - dv, 0.0)

        acc = lax.fori_loop(0, nfull, sum_c,
                            jnp.zeros((_LANES,), jnp.float32))
        acc = acc + jnp.maximum(xt - dv, 0.0)
        tot = _bcast_last(plsc.cumsum(acc))
        inv = jnp.full((_LANES,), 1.0, jnp.float32) / (
            tot + jnp.full((_LANES,), _EPS, jnp.float32))

        tvec_buf[...] = jnp.where(lane_iota == 0, dv, inv)
        pltpu.sync_copy(tvec_buf, t_hbm.at[row])

    cp = pltpu.async_copy(x_hbm.at[row0], bufs[0], sems[0])
    for r in range(_ROWS_PER_W):
        nxt = None
        if r + 1 < _ROWS_PER_W:
            nxt = pltpu.async_copy(
                x_hbm.at[row0 + r + 1], bufs[(r + 1) % 2], sems[(r + 1) % 2])
        cp.wait()
        process(row0 + r, bufs[r % 2])
        cp = nxt


_TC_BLOCK = 32


def _tc_norm_body(x_ref, t_ref, o_ref):
    x = x_ref[...]
    t = t_ref[...]
    d = t[:, :1]
    inv = t[:, 1:2]
    o_ref[...] = jnp.maximum(x - d, 0.0) * inv


def kernel(attn_raw):
    t = _sc_topk(attn_raw)
    out = pl.pallas_call(
        _tc_norm_body,
        grid=(_ROWS // _TC_BLOCK,),
        in_specs=[
            pl.BlockSpec((_TC_BLOCK, _COLS), lambda i: (i, 0)),
            pl.BlockSpec((_TC_BLOCK, _LANES), lambda i: (i, 0)),
        ],
        out_specs=pl.BlockSpec((_TC_BLOCK, _COLS), lambda i: (i, 0)),
        out_shape=jax.ShapeDtypeStruct((_ROWS, _COLS), jnp.float32),
    )(attn_raw, t)
    return out


# speedup vs baseline: 33.6750x; 1.0636x over previous
import functools

import jax
import jax.numpy as jnp
from jax import lax
from jax.experimental import pallas as pl
from jax.experimental.pallas import tpu as pltpu
from jax.experimental.pallas import tpu_sc as plsc

_EPS = 1e-7
_ROWS = 128
_COLS = 32768
_LANES = 16
_NBLK = _COLS // _LANES
_NWORKERS = 32
_ROWS_PER_W = _ROWS // _NWORKERS
_NEG_INF = float("-inf")


def _sortd(v):
    k, _ = plsc.sort_key_val(v, v, descending=True)
    return k


def _tophalf(a, b):
    return _sortd(jnp.maximum(a, lax.rev(b, (0,))))


def _bothhalves(a, b):
    br = lax.rev(b, (0,))
    return _sortd(jnp.maximum(a, br)), _sortd(jnp.minimum(a, br))


_GATHER_DN = lax.GatherDimensionNumbers(
    offset_dims=(), collapsed_slice_dims=(0,), start_index_map=(0,))


def _bcast_last(v):
    idx = jnp.full((_LANES, 1), _LANES - 1, jnp.int32)
    return lax.gather(v, idx, _GATHER_DN, (1,),
                      mode=lax.GatherScatterMode.PROMISE_IN_BOUNDS)


_SG = 8
_NSG = _NBLK // _SG


def _merge_block(x, s0, s1):
    h = _tophalf(x, s1)
    return _bothhalves(s0, h)


@functools.partial(
    pl.kernel,
    out_type=jax.ShapeDtypeStruct((_ROWS, _COLS), jnp.float32),
    mesh=plsc.VectorSubcoreMesh(core_axis_name="c", subcore_axis_name="s"),
    compiler_params=pltpu.CompilerParams(needs_layout_passes=False),
    scratch_types=[
        pltpu.VMEM((_COLS,), jnp.float32),
        pltpu.VMEM((_COLS,), jnp.float32),
        pltpu.VMEM((_COLS + _LANES,), jnp.float32),
        pltpu.VMEM((_NSG * _LANES,), jnp.float32),
        pltpu.VMEM((_NSG + _LANES,), jnp.int32),
        pltpu.SemaphoreType.DMA,
        pltpu.SemaphoreType.DMA,
        pltpu.SemaphoreType.DMA,
        pltpu.SemaphoreType.DMA,
    ],
)
def _sc_topk_norm(x_hbm, o_hbm, row_a, row_b, cand_buf, sgmax_buf, wl_buf,
                  sem_a, sem_b, sem_oa, sem_ob):
    wid = lax.axis_index("s") * 2 + lax.axis_index("c")
    lane_iota = lax.iota(jnp.int32, _LANES)
    row0 = wid * _ROWS_PER_W
    bufs = [row_a, row_b]
    sems = [sem_a, sem_b]
    osems = [sem_oa, sem_ob]
    ninf = jnp.full((_LANES,), _NEG_INF, jnp.float32)
    zero = jnp.zeros((_LANES,), jnp.int32)

    def process(row, row_buf, osem):
        def pass_a(i, ms):
            m0, m1 = ms
            base = i * (_SG * _LANES)
            v = [row_buf[pl.ds(base + j * _LANES, _LANES)]
                 for j in range(_SG)]
            e = jnp.maximum(jnp.maximum(v[0], v[2]), jnp.maximum(v[4], v[6]))
            o = jnp.maximum(jnp.maximum(v[1], v[3]), jnp.maximum(v[5], v[7]))
            sgmax_buf[pl.ds(i * _LANES, _LANES)] = jnp.maximum(e, o)
            return jnp.maximum(m0, e), jnp.maximum(m1, o)

        m0, m1 = lax.fori_loop(0, _NSG, pass_a, (ninf, ninf))
        g = _bcast_last(_sortd(jnp.minimum(m0, m1)))

        woff = zero
        for v in range(_NSG // _LANES):
            sgm = sgmax_buf[pl.ds(v * _LANES, _LANES)]
            m = sgm >= g
            pos = plsc.cumsum(jnp.where(m, 1, 0).astype(jnp.int32))
            plsc.store_scatter(wl_buf, [woff + pos - 1],
                               lane_iota + v * _LANES, mask=m)
            woff = woff + plsc.all_reduce_population_count(m)
        n_sg = jnp.max(woff)

        def pass_b(i, coff):
            wlv = wl_buf[pl.ds(i, _LANES)]
            base = wlv[0] * (_SG * _LANES)
            for j in range(_SG):
                x = row_buf[pl.ds(base + j * _LANES, _LANES)]
                m = x >= g
                pos = plsc.cumsum(jnp.where(m, 1, 0).astype(jnp.int32))
                plsc.store_scatter(cand_buf, [coff + pos - 1], x, mask=m)
                coff = coff + plsc.all_reduce_population_count(m)
            return coff

        coff = lax.fori_loop(0, n_sg, pass_b, zero)
        ncand = jnp.max(coff)
        nfull = ncand // _LANES

        def pass_c(k, st):
            s0, s1 = st
            x = _sortd(cand_buf[pl.ds(k * _LANES, _LANES)])
            return _merge_block(x, s0, s1)

        s0, s1 = lax.fori_loop(0, nfull, pass_c, (ninf, ninf))
        xt = cand_buf[pl.ds(nfull * _LANES, _LANES)]
        xt = jnp.where(lane_iota < (ncand - nfull * _LANES), xt, ninf)
        s0, s1 = _merge_block(_sortd(xt), s0, s1)

        dv = _bcast_last(s1) + jnp.full((_LANES,), _EPS, jnp.float32)

        def sum_c(k, acc):
            c = cand_buf[pl.ds(k * _LANES, _LANES)]
            return acc + jnp.maximum(c - dv, 0.0)

        acc = lax.fori_loop(0, nfull, sum_c,
                            jnp.zeros((_LANES,), jnp.float32))
        acc = acc + jnp.maximum(xt - dv, 0.0)
        tot = _bcast_last(plsc.cumsum(acc))
        inv = jnp.full((_LANES,), 1.0, jnp.float32) / (
            tot + jnp.full((_LANES,), _EPS, jnp.float32))

        def pass_d(i, carry):
            base = i * (_SG * _LANES)
            for j in range(_SG):
                sl = pl.ds(base + j * _LANES, _LANES)
                row_buf[sl] = jnp.maximum(row_buf[sl] - dv, 0.0) * inv
            return carry

        lax.fori_loop(0, _NSG, pass_d, 0)
        return pltpu.async_copy(row_buf, o_hbm.at[row], osem)

    out_cps = [None, None]
    cp = pltpu.async_copy(x_hbm.at[row0], bufs[0], sems[0])
    for r in range(_ROWS_PER_W):
        nxt = None
        if r + 1 < _ROWS_PER_W:
            b = (r + 1) % 2
            if out_cps[b] is not None:
                out_cps[b].wait()
                out_cps[b] = None
            nxt = pltpu.async_copy(x_hbm.at[row0 + r + 1], bufs[b], sems[b])
        cp.wait()
        out_cps[r % 2] = process(row0 + r, bufs[r % 2], osems[r % 2])
        cp = nxt
    for ocp in out_cps:
        if ocp is not None:
            ocp.wait()


def kernel(attn_raw):
    return _sc_topk_norm(attn_raw)
